# baseline (speedup 1.0000x reference)
import functools

import jax
import jax.numpy as jnp
from jax import lax
from jax.experimental import pallas as pl
from jax.experimental.pallas import tpu as pltpu
from jax.experimental.pallas import tpu_sc as plsc

SEN = 10
ROWS = 16384
EMB = 64
B = ROWS * SEN

NC = 2
NS = 16
NW = NC * NS
SENT_PER_W = ROWS // NW
SHI_PER_W = SENT_PER_W // 128


def _body(ids_hbm, table_hbm, out_hbm, len_hbm, idx_v, rows0_v, rows1_v,
          tbuf_v, len_v, gsem0, gsem1, ssem):
    wid = lax.axis_index("s") * NC + lax.axis_index("c")
    sb = wid * SENT_PER_W
    shi0 = wid * SHI_PER_W
    bufs = (rows0_v, rows1_v)
    gsems = (gsem0, gsem1)
    lane = lax.iota(jnp.int32, 16)

    for j in range(SEN):
        pltpu.sync_copy(
            ids_hbm.at[pl.ds(j, 1), pl.ds(sb, SENT_PER_W)],
            idx_v.at[pl.ds(j, 1)],
        )

    def gather(j, buf):
        return pltpu.async_copy(
            table_hbm.at[idx_v.at[j]], buf, gsems[j % 2]
        )

    gcp = gather(0, bufs[0])

    def len_block(k, carry):
        cnt = jnp.zeros((16,), jnp.int32)
        for j in range(SEN):
            v = idx_v[j, pl.ds(k * 16, 16)]
            cnt = cnt + jnp.minimum(v, 1)
        len_v[pl.ds(k * 16, 16)] = jnp.maximum(cnt, 1).astype(jnp.float32)
        return carry

    lax.fori_loop(0, SENT_PER_W // 16, len_block, 0)
    pltpu.sync_copy(len_v, len_hbm.at[pl.ds(sb, SENT_PER_W)])

    scps = []
    for j in range(SEN):
        b = j % 2
        if j + 1 < SEN:
            ngcp = gather(j + 1, bufs[1 - b])
        gcp.wait()
        for scp in scps:
            scp.wait()
        scps = []
        rows = bufs[b]

        def trans_block(i, carry):
            e_hi = i >> 2
            s_hi_l = i & 3
            for e_lo in range(8):
                e = e_hi * 8 + e_lo
                for k in range(8):
                    svec = s_hi_l * 128 + k * 16 + lane
                    v = plsc.load_gather(rows, [svec, jnp.full((16,), e, jnp.int32)])
                    tbuf_v[e_hi, s_hi_l, e_lo, pl.ds(k * 16, 16)] = v
            return carry

        lax.fori_loop(0, 8 * SHI_PER_W, trans_block, 0)

        for e_hi in range(8):
            scps.append(
                pltpu.async_copy(
                    tbuf_v.at[pl.ds(e_hi, 1)],
                    out_hbm.at[j, pl.ds(e_hi, 1), pl.ds(shi0, SHI_PER_W)],
                    ssem,
                )
            )
        if j + 1 < SEN:
            gcp = ngcp
    for scp in scps:
        scp.wait()


@jax.jit
def _emb_lookup(ids_t, table):
    mesh = plsc.VectorSubcoreMesh(core_axis_name="c", subcore_axis_name="s")
    return pl.kernel(
        _body,
        out_type=(
            jax.ShapeDtypeStruct((SEN, 8, 128, 8, 128), jnp.float32),
            jax.ShapeDtypeStruct((ROWS,), jnp.float32),
        ),
        mesh=mesh,
        compiler_params=pltpu.CompilerParams(
            needs_layout_passes=False, use_tc_tiling_on_sc=False
        ),
        scratch_types=[
            pltpu.VMEM((SEN, SENT_PER_W), jnp.int32),
            pltpu.VMEM((SENT_PER_W, EMB), jnp.float32),
            pltpu.VMEM((SENT_PER_W, EMB), jnp.float32),
            pltpu.VMEM((8, SHI_PER_W, 8, 128), jnp.float32),
            pltpu.VMEM((SENT_PER_W,), jnp.float32),
            pltpu.SemaphoreType.DMA,
            pltpu.SemaphoreType.DMA,
            pltpu.SemaphoreType.DMA,
        ],
    )(ids_t, table)


def kernel(ids, table):
    ids_t = ids.astype(jnp.int32).T
    out5, length = _emb_lookup(ids_t, table)
    emb = out5.transpose(2, 4, 0, 1, 3).reshape(ROWS, SEN, EMB)
    return emb, length


# speedup vs baseline: 1.1782x; 1.1782x over previous
import functools

import jax
import jax.numpy as jnp
from jax import lax
from jax.experimental import pallas as pl
from jax.experimental.pallas import tpu as pltpu
from jax.experimental.pallas import tpu_sc as plsc

SEN = 10
ROWS = 16384
EMB = 64
B = ROWS * SEN

NC = 2
NS = 16
NW = NC * NS
SENT_PER_W = ROWS // NW
SHI_PER_W = SENT_PER_W // 128


def _body(ids_hbm, table_hbm, out_hbm, len_hbm, idx_v, rows0_v, rows1_v,
          tbuf_v, len_v, gsem0, gsem1, ssem):
    wid = lax.axis_index("s") * NC + lax.axis_index("c")
    sb = wid * SENT_PER_W
    shi0 = wid * SHI_PER_W
    bufs = (rows0_v, rows1_v)
    gsems = (gsem0, gsem1)
    lane = lax.iota(jnp.int32, 16)

    for j in range(SEN):
        pltpu.sync_copy(
            ids_hbm.at[pl.ds(j, 1), pl.ds(sb, SENT_PER_W)],
            idx_v.at[pl.ds(j, 1)],
        )

    def gather(j, buf):
        return pltpu.async_copy(
            table_hbm.at[idx_v.at[j]], buf, gsems[j % 2]
        )

    gcp = gather(0, bufs[0])

    def len_block(k, carry):
        cnt = jnp.zeros((16,), jnp.int32)
        for j in range(SEN):
            v = idx_v[j, pl.ds(k * 16, 16)]
            cnt = cnt + jnp.minimum(v, 1)
        len_v[pl.ds(k * 16, 16)] = jnp.maximum(cnt, 1).astype(jnp.float32)
        return carry

    lax.fori_loop(0, SENT_PER_W // 16, len_block, 0)
    pltpu.sync_copy(len_v, len_hbm.at[pl.ds(sb, SENT_PER_W)])

    scps = []
    for j in range(SEN):
        b = j % 2
        if j + 1 < SEN:
            ngcp = gather(j + 1, bufs[1 - b])
        gcp.wait()
        for scp in scps:
            scp.wait()
        scps = []
        rows = bufs[b]

        def trans_block(i, carry):
            e_hi = i >> 2
            s_hi_l = i & 3
            svecs = [s_hi_l * 128 + k * 16 + lane for k in range(8)]
            for e_lo in range(8):
                e = e_hi * 8 + e_lo
                ev = jnp.full((16,), e, jnp.int32)
                vs = [plsc.load_gather(rows, [svecs[k], ev]) for k in range(8)]
                for k in range(8):
                    tbuf_v[e_hi, s_hi_l, e_lo, pl.ds(k * 16, 16)] = vs[k]
            return carry

        lax.fori_loop(0, 8 * SHI_PER_W, trans_block, 0)

        for e_hi in range(8):
            scps.append(
                pltpu.async_copy(
                    tbuf_v.at[pl.ds(e_hi, 1)],
                    out_hbm.at[j, pl.ds(e_hi, 1), pl.ds(shi0, SHI_PER_W)],
                    ssem,
                )
            )
        if j + 1 < SEN:
            gcp = ngcp
    for scp in scps:
        scp.wait()


@jax.jit
def _emb_lookup(ids_t, table):
    mesh = plsc.VectorSubcoreMesh(core_axis_name="c", subcore_axis_name="s")
    return pl.kernel(
        _body,
        out_type=(
            jax.ShapeDtypeStruct((SEN, 8, 128, 8, 128), jnp.float32),
            jax.ShapeDtypeStruct((ROWS,), jnp.float32),
        ),
        mesh=mesh,
        compiler_params=pltpu.CompilerParams(
            needs_layout_passes=False, use_tc_tiling_on_sc=False
        ),
        scratch_types=[
            pltpu.VMEM((SEN, SENT_PER_W), jnp.int32),
            pltpu.VMEM((SENT_PER_W, EMB), jnp.float32),
            pltpu.VMEM((SENT_PER_W, EMB), jnp.float32),
            pltpu.VMEM((8, SHI_PER_W, 8, 128), jnp.float32),
            pltpu.VMEM((SENT_PER_W,), jnp.float32),
            pltpu.SemaphoreType.DMA,
            pltpu.SemaphoreType.DMA,
            pltpu.SemaphoreType.DMA,
        ],
    )(ids_t, table)


def kernel(ids, table):
    ids_t = ids.astype(jnp.int32).T
    out5, length = _emb_lookup(ids_t, table)
    emb = out5.transpose(2, 4, 0, 1, 3).reshape(ROWS, SEN, EMB)
    return emb, length


# speedup vs baseline: 1.2112x; 1.0280x over previous
import functools

import jax
import jax.numpy as jnp
from jax import lax
from jax.experimental import pallas as pl
from jax.experimental.pallas import tpu as pltpu
from jax.experimental.pallas import tpu_sc as plsc

SEN = 10
ROWS = 16384
EMB = 64
B = ROWS * SEN

NC = 2
NS = 16
NW = NC * NS
SENT_PER_W = ROWS // NW
SHI_PER_W = SENT_PER_W // 128
HALF = 256
NUNIT = SEN * (SENT_PER_W // HALF)


def _body(ids_hbm, table_hbm, out_hbm, len_hbm, idx_v, rows0_v, rows1_v,
          rows2_v, tb0_v, tb1_v, len_v, gsem0, gsem1, gsem2, ssem0, ssem1):
    wid = lax.axis_index("s") * NC + lax.axis_index("c")
    sb = wid * SENT_PER_W
    shi0 = wid * SHI_PER_W
    rbufs = (rows0_v, rows1_v, rows2_v)
    gsems = (gsem0, gsem1, gsem2)
    tbufs = (tb0_v, tb1_v)
    ssems = (ssem0, ssem1)
    lane = lax.iota(jnp.int32, 16)

    for j in range(SEN):
        pltpu.sync_copy(
            ids_hbm.at[pl.ds(j, 1), pl.ds(sb, SENT_PER_W)],
            idx_v.at[pl.ds(j, 1)],
        )

    def gather(u):
        j, h = u >> 1, u & 1
        return pltpu.async_copy(
            table_hbm.at[idx_v.at[j, pl.ds(h * HALF, HALF)]],
            rbufs[u % 3],
            gsems[u % 3],
        )

    gcps = [gather(0), gather(1)]

    def len_block(k, carry):
        cnt = jnp.zeros((16,), jnp.int32)
        for j in range(SEN):
            v = idx_v[j, pl.ds(k * 16, 16)]
            cnt = cnt + jnp.minimum(v, 1)
        len_v[pl.ds(k * 16, 16)] = jnp.maximum(cnt, 1).astype(jnp.float32)
        return carry

    lax.fori_loop(0, SENT_PER_W // 16, len_block, 0)
    pltpu.sync_copy(len_v, len_hbm.at[pl.ds(sb, SENT_PER_W)])

    scps = [None, None]
    for u in range(NUNIT):
        j, h = u >> 1, u & 1
        if u + 2 < NUNIT:
            gcps.append(gather(u + 2))
        gcps[u].wait()
        rows = rbufs[u % 3]
        tb = tbufs[u % 2]
        if scps[u % 2] is not None:
            for scp in scps[u % 2]:
                scp.wait()

        def trans_block(i, carry):
            e_hi = i >> 1
            s_hi_l = i & 1
            svecs = [s_hi_l * 128 + k * 16 + lane for k in range(8)]
            for e_lo in range(8):
                e = e_hi * 8 + e_lo
                ev = jnp.full((16,), e, jnp.int32)
                vs = [plsc.load_gather(rows, [svecs[k], ev]) for k in range(8)]
                for k in range(8):
                    tb[e_hi, s_hi_l, e_lo, pl.ds(k * 16, 16)] = vs[k]
            return carry

        lax.fori_loop(0, 16, trans_block, 0)

        scps[u % 2] = [
            pltpu.async_copy(
                tb.at[e_hi],
                out_hbm.at[j, e_hi, pl.ds(shi0 + h * 2, 2)],
                ssems[u % 2],
            )
            for e_hi in range(8)
        ]
    for pair in scps:
        for scp in pair:
            scp.wait()


@jax.jit
def _emb_lookup(ids_t, table):
    mesh = plsc.VectorSubcoreMesh(core_axis_name="c", subcore_axis_name="s")
    return pl.kernel(
        _body,
        out_type=(
            jax.ShapeDtypeStruct((SEN, 8, 128, 8, 128), jnp.float32),
            jax.ShapeDtypeStruct((ROWS,), jnp.float32),
        ),
        mesh=mesh,
        compiler_params=pltpu.CompilerParams(
            needs_layout_passes=False, use_tc_tiling_on_sc=False
        ),
        scratch_types=[
            pltpu.VMEM((SEN, SENT_PER_W), jnp.int32),
            pltpu.VMEM((HALF, EMB), jnp.float32),
            pltpu.VMEM((HALF, EMB), jnp.float32),
            pltpu.VMEM((HALF, EMB), jnp.float32),
            pltpu.VMEM((8, 2, 8, 128), jnp.float32),
            pltpu.VMEM((8, 2, 8, 128), jnp.float32),
            pltpu.VMEM((SENT_PER_W,), jnp.float32),
            pltpu.SemaphoreType.DMA,
            pltpu.SemaphoreType.DMA,
            pltpu.SemaphoreType.DMA,
            pltpu.SemaphoreType.DMA,
            pltpu.SemaphoreType.DMA,
        ],
    )(ids_t, table)


def kernel(ids, table):
    ids_t = ids.astype(jnp.int32).T
    out5, length = _emb_lookup(ids_t, table)
    emb = out5.transpose(2, 4, 0, 1, 3).reshape(ROWS, SEN, EMB)
    return emb, length


# speedup vs baseline: 1.4838x; 1.2251x over previous
import functools

import jax
import jax.numpy as jnp
from jax import lax
from jax.experimental import pallas as pl
from jax.experimental.pallas import tpu as pltpu
from jax.experimental.pallas import tpu_sc as plsc

SEN = 10
ROWS = 16384
EMB = 64
B = ROWS * SEN

NC = 2
NS = 16
NW = NC * NS
B_PER_W = B // NW
SENT_PER_W = ROWS // NW
CHUNK = 640
NCHUNK = B_PER_W // CHUNK
GROUPS_PER_CHUNK = (SENT_PER_W // 16) // NCHUNK


def _lengths_for_chunk(c, idx_v, len_v):
    def sent_group(g, carry):
        lvec = (c * GROUPS_PER_CHUNK + g) * 16 + lax.iota(jnp.int32, 16)
        pos0 = lvec * SEN
        cnt = jnp.zeros((16,), jnp.int32)
        for j in range(SEN):
            v = plsc.load_gather(idx_v, [pos0 + j])
            cnt = cnt + jnp.minimum(v, 1)
        n = jnp.maximum(cnt, 1).astype(jnp.float32)
        len_v[pl.ds((c * GROUPS_PER_CHUNK + g) * 16, 16)] = n
        return carry

    lax.fori_loop(0, GROUPS_PER_CHUNK, sent_group, 0)


def _body(ids_hbm, table_hbm, out_hbm, len_hbm, idx_v, rows0_v, rows1_v,
          len_v, gsem0, gsem1, ssem0, ssem1):
    wid = lax.axis_index("s") * NC + lax.axis_index("c")
    base = wid * B_PER_W
    sbase = wid * SENT_PER_W
    bufs = (rows0_v, rows1_v)
    gsems = (gsem0, gsem1)
    ssems = (ssem0, ssem1)

    pltpu.sync_copy(ids_hbm.at[pl.ds(base, B_PER_W)], idx_v)

    def gather(c):
        return pltpu.async_copy(
            table_hbm.at[idx_v.at[pl.ds(c * CHUNK, CHUNK)]],
            bufs[c % 2],
            gsems[c % 2],
        )

    gcp = gather(0)
    scp = None
    for c in range(NCHUNK):
        b = c % 2
        if scp is not None:
            scp.wait()
        ngcp = gather(c + 1) if c + 1 < NCHUNK else None
        _lengths_for_chunk(c, idx_v, len_v)
        gcp.wait()
        scp = pltpu.async_copy(
            bufs[b], out_hbm.at[pl.ds(base + c * CHUNK, CHUNK)], ssems[b]
        )
        gcp = ngcp
    scp.wait()
    pltpu.sync_copy(len_v, len_hbm.at[pl.ds(sbase, SENT_PER_W)])


@jax.jit
def _emb_lookup(ids_flat, table):
    mesh = plsc.VectorSubcoreMesh(core_axis_name="c", subcore_axis_name="s")
    return pl.kernel(
        _body,
        out_type=(
            jax.ShapeDtypeStruct((B, EMB), jnp.float32),
            jax.ShapeDtypeStruct((ROWS,), jnp.float32),
        ),
        mesh=mesh,
        compiler_params=pltpu.CompilerParams(
            needs_layout_passes=False, use_tc_tiling_on_sc=False
        ),
        scratch_types=[
            pltpu.VMEM((B_PER_W,), jnp.int32),
            pltpu.VMEM((CHUNK, EMB), jnp.float32),
            pltpu.VMEM((CHUNK, EMB), jnp.float32),
            pltpu.VMEM((SENT_PER_W,), jnp.float32),
            pltpu.SemaphoreType.DMA,
            pltpu.SemaphoreType.DMA,
            pltpu.SemaphoreType.DMA,
            pltpu.SemaphoreType.DMA,
        ],
    )(ids_flat, table)


def kernel(ids, table):
    ids_flat = ids.astype(jnp.int32).reshape(B)
    emb_flat, length = _emb_lookup(ids_flat, table)
    return emb_flat.reshape(ROWS, SEN, EMB), length
